# baseline (speedup 1.0000x reference)
import jax
import jax.numpy as jnp
from jax import lax
from jax.experimental import pallas as pl
from jax.experimental.pallas import tpu as pltpu
from jax.experimental.pallas import tpu_sc as plsc

D = 1024
SEQ = 4096
BATCH = 4
LANES = 16
NC = 2
NS = 16
NW = NC * NS
POS_PER_W = SEQ // NW
CHUNK = 8
N_CHUNKS = POS_PER_W // CHUNK
NG = 3
VREGS_PER_ROW = D // LANES


def _emb_body(ids_hbm, tok_hbm, pos_hbm, out_hbm,
              idx_v, *bufs_and_sems):
    toks = bufs_and_sems[:NG * BATCH]
    poss = bufs_and_sems[NG * BATCH:NG * BATCH + NG]
    gsems = bufs_and_sems[NG * BATCH + NG:NG * BATCH + 2 * NG]
    wsems = bufs_and_sems[NG * BATCH + 2 * NG:]

    wid = lax.axis_index("s") * NC + lax.axis_index("c")
    pos0 = wid * POS_PER_W
    for b in range(BATCH):
        pltpu.sync_copy(ids_hbm.at[pl.ds(b * SEQ + pos0, POS_PER_W)],
                        idx_v.at[pl.ds(b * POS_PER_W, POS_PER_W)])

    def gather_desc(c, g, b):
        sl = pl.ds(b * POS_PER_W + c * CHUNK, CHUNK)
        return pltpu.make_async_copy(tok_hbm.at[idx_v.at[sl]],
                                     toks[g * BATCH + b], gsems[g])

    def pos_desc(c, g):
        sl = pl.ds(pos0 + c * CHUNK, CHUNK)
        return pltpu.make_async_copy(pos_hbm.at[sl], poss[g], gsems[g])

    def wb_desc(c, g, b):
        sl = pl.ds(b * SEQ + pos0 + c * CHUNK, CHUNK)
        return pltpu.make_async_copy(toks[g * BATCH + b],
                                     out_hbm.at[sl], wsems[g])

    def do_chunk(c, g, wait_prev_wb, issue_next):
        gn = (g + 1) % NG
        for b in range(BATCH):
            gather_desc(c, g, b).wait()
        if issue_next:
            if wait_prev_wb:
                for b in range(BATCH):
                    wb_desc(c - 2, gn, b).wait()
            for b in range(BATCH):
                gather_desc(c + 1, gn, b).start()
            pos_desc(c + 1, gn).start()
        pos_desc(c, g).wait()

        def add_row(r, c2):
            for l in range(VREGS_PER_ROW):
                sl = pl.ds(l * LANES, LANES)
                pv = poss[g][r, sl]
                for b in range(BATCH):
                    plsc.addupdate(toks[g * BATCH + b].at[r, sl], pv)
            return c2

        lax.fori_loop(0, CHUNK, add_row, 0)
        for b in range(BATCH):
            wb_desc(c, g, b).start()

    for b in range(BATCH):
        gather_desc(0, 0, b).start()
    for b in range(BATCH):
        gather_desc(1, 1, b).start()
    pos_desc(0, 0).start()
    pos_desc(1, 1).start()
    do_chunk(0, 0, wait_prev_wb=False, issue_next=False)
    do_chunk(1, 1, wait_prev_wb=False, issue_next=True)

    def triple_body(p, carry):
        c0 = 3 * p + 2
        do_chunk(c0, 2, wait_prev_wb=True, issue_next=True)
        do_chunk(c0 + 1, 0, wait_prev_wb=True, issue_next=True)
        do_chunk(c0 + 2, 1, wait_prev_wb=True, issue_next=True)
        return carry

    lax.fori_loop(0, (N_CHUNKS - 4) // 3, triple_body, 0)

    do_chunk(N_CHUNKS - 2, (N_CHUNKS - 2) % NG, wait_prev_wb=True,
             issue_next=True)
    do_chunk(N_CHUNKS - 1, (N_CHUNKS - 1) % NG, wait_prev_wb=False,
             issue_next=False)
    for c in (N_CHUNKS - 3, N_CHUNKS - 2, N_CHUNKS - 1):
        for b in range(BATCH):
            wb_desc(c, c % NG, b).wait()


def kernel(input_ids, token_table, pos_table):
    b, s = input_ids.shape
    ids_flat = input_ids.reshape(-1).astype(jnp.int32)
    scratch = [pltpu.VMEM((BATCH * POS_PER_W,), jnp.int32)]
    scratch += [pltpu.VMEM((CHUNK, D), jnp.float32) for _ in range(NG * BATCH)]
    scratch += [pltpu.VMEM((CHUNK, D), jnp.float32) for _ in range(NG)]
    scratch += [pltpu.SemaphoreType.DMA for _ in range(2 * NG)]
    k = pl.kernel(
        _emb_body,
        mesh=plsc.VectorSubcoreMesh(core_axis_name="c", subcore_axis_name="s"),
        out_type=jax.ShapeDtypeStruct((b * s, D), jnp.float32),
        scratch_types=scratch,
    )
    out = k(ids_flat, token_table, pos_table)
    return out.reshape(b, s, D)


# speedup vs baseline: 1.0171x; 1.0171x over previous
import jax
import jax.numpy as jnp
from jax import lax
from jax.experimental import pallas as pl
from jax.experimental.pallas import tpu as pltpu
from jax.experimental.pallas import tpu_sc as plsc

D = 1024
SEQ = 4096
BATCH = 4
LANES = 16
NC = 2
NS = 16
NW = NC * NS
POS_PER_W = SEQ // NW
CHUNK = 8
N_CHUNKS = POS_PER_W // CHUNK
NG = 3
VREGS_PER_ROW = D // LANES


def _emb_body(ids_hbm, tok_hbm, pos_hbm, out_hbm,
              idx_v, *bufs_and_sems):
    toks = bufs_and_sems[:NG * BATCH]
    poss = bufs_and_sems[NG * BATCH:NG * BATCH + NG]
    gsems = bufs_and_sems[NG * BATCH + NG:NG * BATCH + 2 * NG]
    wsems = bufs_and_sems[NG * BATCH + 2 * NG:]

    wid = lax.axis_index("s") * NC + lax.axis_index("c")
    pos0 = wid * POS_PER_W
    idescs = [pltpu.make_async_copy(ids_hbm.at[pl.ds(b * SEQ + pos0, POS_PER_W)],
                                    idx_v.at[pl.ds(b * POS_PER_W, POS_PER_W)],
                                    wsems[0])
              for b in range(BATCH)]
    for d in idescs:
        d.start()
    for d in idescs:
        d.wait()

    def gather_desc(c, g, b):
        sl = pl.ds(b * POS_PER_W + c * CHUNK, CHUNK)
        return pltpu.make_async_copy(tok_hbm.at[idx_v.at[sl]],
                                     toks[g * BATCH + b], gsems[g])

    def pos_desc(c, g):
        sl = pl.ds(pos0 + c * CHUNK, CHUNK)
        return pltpu.make_async_copy(pos_hbm.at[sl], poss[g], gsems[g])

    def wb_desc(c, g, b):
        sl = pl.ds(b * SEQ + pos0 + c * CHUNK, CHUNK)
        return pltpu.make_async_copy(toks[g * BATCH + b],
                                     out_hbm.at[sl], wsems[g])

    def do_chunk(c, g, wait_prev_wb, issue_next):
        gn = (g + 1) % NG
        for b in range(BATCH):
            gather_desc(c, g, b).wait()
        if issue_next:
            if wait_prev_wb:
                for b in range(BATCH):
                    wb_desc(c - 2, gn, b).wait()
            for b in range(BATCH):
                gather_desc(c + 1, gn, b).start()
            pos_desc(c + 1, gn).start()
        pos_desc(c, g).wait()

        def add_row(r, c2):
            for l in range(VREGS_PER_ROW):
                sl = pl.ds(l * LANES, LANES)
                pv = poss[g][r, sl]
                for b in range(BATCH):
                    plsc.addupdate(toks[g * BATCH + b].at[r, sl], pv)
            return c2

        lax.fori_loop(0, CHUNK, add_row, 0)
        for b in range(BATCH):
            wb_desc(c, g, b).start()

    for b in range(BATCH):
        gather_desc(0, 0, b).start()
    for b in range(BATCH):
        gather_desc(1, 1, b).start()
    pos_desc(0, 0).start()
    pos_desc(1, 1).start()
    do_chunk(0, 0, wait_prev_wb=False, issue_next=False)
    do_chunk(1, 1, wait_prev_wb=False, issue_next=True)

    def triple_body(p, carry):
        c0 = 3 * p + 2
        do_chunk(c0, 2, wait_prev_wb=True, issue_next=True)
        do_chunk(c0 + 1, 0, wait_prev_wb=True, issue_next=True)
        do_chunk(c0 + 2, 1, wait_prev_wb=True, issue_next=True)
        return carry

    lax.fori_loop(0, (N_CHUNKS - 4) // 3, triple_body, 0)

    do_chunk(N_CHUNKS - 2, (N_CHUNKS - 2) % NG, wait_prev_wb=True,
             issue_next=True)
    do_chunk(N_CHUNKS - 1, (N_CHUNKS - 1) % NG, wait_prev_wb=False,
             issue_next=False)
    for c in (N_CHUNKS - 3, N_CHUNKS - 2, N_CHUNKS - 1):
        for b in range(BATCH):
            wb_desc(c, c % NG, b).wait()


def kernel(input_ids, token_table, pos_table):
    b, s = input_ids.shape
    ids_flat = input_ids.reshape(-1).astype(jnp.int32)
    scratch = [pltpu.VMEM((BATCH * POS_PER_W,), jnp.int32)]
    scratch += [pltpu.VMEM((CHUNK, D), jnp.float32) for _ in range(NG * BATCH)]
    scratch += [pltpu.VMEM((CHUNK, D), jnp.float32) for _ in range(NG)]
    scratch += [pltpu.SemaphoreType.DMA for _ in range(2 * NG)]
    k = pl.kernel(
        _emb_body,
        mesh=plsc.VectorSubcoreMesh(core_axis_name="c", subcore_axis_name="s"),
        out_type=jax.ShapeDtypeStruct((b * s, D), jnp.float32),
        scratch_types=scratch,
    )
    out = k(ids_flat, token_table, pos_table)
    return out.reshape(b, s, D)
